# baseline (speedup 1.0000x reference)
import functools

import jax
import jax.numpy as jnp
from jax import lax
from jax.experimental import pallas as pl
from jax.experimental.pallas import tpu as pltpu
from jax.experimental.pallas import tpu_sc as plsc

_NUM_LABELS = 3
_D = 1024
_B = 16384
_NC = 2
_NS = 16
_NW = _NC * _NS
_BPW = _B // _NW
_CHUNK = 64
_NCHUNK = _BPW // _CHUNK


def _sc_embedding_lookup(ids, table):
    mesh = plsc.VectorSubcoreMesh(core_axis_name="c", subcore_axis_name="s")

    @functools.partial(
        pl.kernel,
        mesh=mesh,
        out_type=jax.ShapeDtypeStruct((_B, _D), jnp.float32),
        scratch_types=[
            pltpu.VMEM((_BPW,), jnp.int32),
            pltpu.VMEM((_CHUNK, _D), jnp.float32),
            pltpu.SemaphoreType.DMA,
        ],
    )
    def k(ids_hbm, table_hbm, out_hbm, idx_v, rows_v, sem):
        wid = lax.axis_index("s") * _NC + lax.axis_index("c")
        base = wid * _BPW
        pltpu.sync_copy(ids_hbm.at[pl.ds(base, _BPW)], idx_v)
        for c in range(_NCHUNK):
            pltpu.async_copy(
                table_hbm.at[idx_v.at[pl.ds(c * _CHUNK, _CHUNK)]],
                rows_v,
                sem,
            ).wait()
            pltpu.sync_copy(rows_v, out_hbm.at[pl.ds(base + c * _CHUNK, _CHUNK)])

    return k(ids, table)


def kernel(sentiment_ids, embedding_table):
    ids = sentiment_ids.astype(jnp.int32)
    return _sc_embedding_lookup(ids, embedding_table.astype(jnp.float32))


# speedup vs baseline: 3.2416x; 3.2416x over previous
import functools

import jax
import jax.numpy as jnp
from jax import lax
from jax.experimental import pallas as pl
from jax.experimental.pallas import tpu as pltpu
from jax.experimental.pallas import tpu_sc as plsc

_NUM_LABELS = 3
_D = 1024
_B = 16384
_NC = 2
_NS = 16
_NW = _NC * _NS
_BPW = _B // _NW
_CHUNK = 32
_NCHUNK = _BPW // _CHUNK
_NBUF = 2
_L = 16


def _sc_embedding_lookup(ids, table_rep):
    mesh = plsc.VectorSubcoreMesh(core_axis_name="c", subcore_axis_name="s")

    @functools.partial(
        pl.kernel,
        mesh=mesh,
        out_type=jax.ShapeDtypeStruct((_B, _D), jnp.float32),
        scratch_types=[
            pltpu.VMEM((_BPW,), jnp.int32),
            pltpu.VMEM((_NBUF, _CHUNK, _D), jnp.float32),
            pltpu.SemaphoreType.DMA,
            pltpu.SemaphoreType.DMA,
            pltpu.SemaphoreType.DMA,
        ],
    )
    def k(ids_hbm, table_hbm, out_hbm, idx_v, rows_v, gsem, osem0, osem1):
        wid = lax.axis_index("s") * _NC + lax.axis_index("c")
        base = wid * _BPW
        pltpu.sync_copy(ids_hbm.at[pl.ds(base, _BPW)], idx_v)
        off = (wid * _NUM_LABELS).astype(jnp.int32)
        for j in range(_BPW // _L):
            sl = pl.ds(j * _L, _L)
            idx_v[sl] = idx_v[sl] + off
        osems = (osem0, osem1)
        out_dma = [None, None]
        for c in range(_NCHUNK):
            p = c % _NBUF
            if out_dma[p] is not None:
                out_dma[p].wait()
            pltpu.async_copy(
                table_hbm.at[idx_v.at[pl.ds(c * _CHUNK, _CHUNK)]],
                rows_v.at[p],
                gsem,
            ).wait()
            out_dma[p] = pltpu.async_copy(
                rows_v.at[p],
                out_hbm.at[pl.ds(base + c * _CHUNK, _CHUNK)],
                osems[p],
            )
        for p in range(_NBUF):
            if out_dma[p] is not None:
                out_dma[p].wait()

    return k(ids, table_rep)


def kernel(sentiment_ids, embedding_table):
    ids = sentiment_ids.astype(jnp.int32)
    table_rep = jnp.tile(embedding_table.astype(jnp.float32), (_NW, 1))
    return _sc_embedding_lookup(ids, table_rep)
